# baseline (device time: 78811 ns/iter reference)
import os

import jax
import jax.numpy as jnp
from jax import lax
from jax.experimental import pallas as pl
from jax.experimental.pallas import tpu as pltpu

N_DEV = 8
H = 2
KVAR = os.environ.get("KVAR", "full")


def kernel(x, w_mat, scale_x, scale_w):
    m_per, k = x.shape
    _, n_total = w_mat.shape
    n_per = n_total // N_DEV
    wc = n_per // H

    def body(x_ref, w_ref, sx_ref, sw_ref, out_ref,
             wbuf, send_q, recv_q, send_s, recv_s, stage,
             wsems, osems, qsend_sems, qrecv_sems, ssend_sems, srecv_sems):
        my = lax.axis_index("i")

        scale = sx_ref[0] * sw_ref[0]
        x_q = x_ref[...].astype(jnp.float8_e4m3fn)

        order = list(range(1, N_DEV)) + [0]
        steps = [(d, h) for d in order for h in range(H)]

        out_dmas = {}

        def flush_block(slot, row_dev):
            dma = pltpu.make_async_copy(
                stage.at[slot],
                out_ref.at[pl.ds(row_dev * m_per, m_per)],
                osems.at[slot],
            )
            dma.start()
            out_dmas[slot] = dma

        def reuse_slot(slot):
            if slot in out_dmas:
                out_dmas.pop(slot).wait()

        def w_copy(d, h, slot):
            tgt = lax.rem(my + d, N_DEV)
            return pltpu.make_async_copy(
                w_ref.at[:, pl.ds(tgt * n_per + h * wc, wc)],
                wbuf.at[slot],
                wsems.at[slot],
            )

        def make_rdmas(d):
            tgt = lax.rem(my + d, N_DEV)
            data = pltpu.make_async_remote_copy(
                src_ref=send_q.at[d - 1],
                dst_ref=recv_q.at[d - 1],
                send_sem=qsend_sems.at[d - 1],
                recv_sem=qrecv_sems.at[d - 1],
                device_id=(tgt,),
                device_id_type=pl.DeviceIdType.MESH,
            )
            scl = pltpu.make_async_remote_copy(
                src_ref=send_s.at[d - 1],
                dst_ref=recv_s.at[d - 1],
                send_sem=ssend_sems.at[d - 1],
                recv_sem=srecv_sems.at[d - 1],
                device_id=(tgt,),
                device_id_type=pl.DeviceIdType.MESH,
            )
            return data, scl

        def process_recv(d):
            src_dev = lax.rem(my - d + N_DEV, N_DEV)
            data, scl = make_rdmas(d)
            data.wait_recv()
            scl.wait_recv()
            slot = d % 2
            reuse_slot(slot)
            stage[slot] = recv_q[d - 1].astype(jnp.float32) * recv_s[d - 1]
            flush_block(slot, src_dev)

        rdmas = {}
        if KVAR != "nocompute":
            w_copy(*steps[0], 0).start()
            w_copy(*steps[1], 1).start()
            for si, (d, h) in enumerate(steps):
                slot = si % 3
                if si + 2 < len(steps):
                    w_copy(*steps[si + 2], (si + 2) % 3).start()
                w_copy(d, h, slot).wait()
                wj = wbuf[slot].astype(jnp.float8_e4m3fn)
                acc = lax.dot_general(
                    x_q, wj, (((1,), (0,)), ((), ())),
                    preferred_element_type=jnp.float32)
                y = acc * scale
                z = y * (1.0 / (1.0 + jnp.exp(-jnp.clip(y, -60.0, 60.0))))
                if d == 0:
                    if h == 0:
                        reuse_slot(0)
                    stage[0, :, pl.ds(h * wc, wc)] = z
                    if h == H - 1:
                        flush_block(0, my)
                else:
                    s_h = jnp.maximum(
                        jnp.max(jnp.abs(z), axis=0, keepdims=True),
                        1e-30) * (1.0 / 127.0)
                    q = jnp.clip(jnp.round(z * (1.0 / s_h)), -127.0, 127.0)
                    send_q[d - 1, :, pl.ds(h * wc, wc)] = q.astype(jnp.int8)
                    send_s[d - 1, :, pl.ds(h * wc, wc)] = s_h
                    if h == H - 1 and KVAR == "full":
                        data, scl = make_rdmas(d)
                        data.start()
                        scl.start()
                        rdmas[d] = (data, scl)
                        if d >= 2:
                            process_recv(d - 1)
        else:
            for d in range(1, N_DEV):
                data, scl = make_rdmas(d)
                data.start()
                scl.start()
                rdmas[d] = (data, scl)

        if KVAR != "nocomm":
            if KVAR == "full":
                process_recv(N_DEV - 1)
            else:
                for d in range(1, N_DEV):
                    process_recv(d)
            for d in rdmas:
                rdmas[d][0].wait_send()
                rdmas[d][1].wait_send()
        for slot in list(out_dmas):
            out_dmas.pop(slot).wait()

    return pl.pallas_call(
        body,
        out_shape=jax.ShapeDtypeStruct((N_DEV * m_per, n_per), jnp.float32),
        in_specs=[
            pl.BlockSpec(memory_space=pltpu.VMEM),
            pl.BlockSpec(memory_space=pl.ANY),
            pl.BlockSpec(memory_space=pltpu.SMEM),
            pl.BlockSpec(memory_space=pltpu.SMEM),
        ],
        out_specs=pl.BlockSpec(memory_space=pl.ANY),
        scratch_shapes=[
            pltpu.VMEM((3, k, wc), jnp.float32),
            pltpu.VMEM((N_DEV - 1, m_per, n_per), jnp.int8),
            pltpu.VMEM((N_DEV - 1, m_per, n_per), jnp.int8),
            pltpu.VMEM((N_DEV - 1, 1, n_per), jnp.float32),
            pltpu.VMEM((N_DEV - 1, 1, n_per), jnp.float32),
            pltpu.VMEM((2, m_per, n_per), jnp.float32),
            pltpu.SemaphoreType.DMA((3,)),
            pltpu.SemaphoreType.DMA((2,)),
            pltpu.SemaphoreType.DMA((N_DEV - 1,)),
            pltpu.SemaphoreType.DMA((N_DEV - 1,)),
            pltpu.SemaphoreType.DMA((N_DEV - 1,)),
            pltpu.SemaphoreType.DMA((N_DEV - 1,)),
        ],
        compiler_params=pltpu.CompilerParams(
            vmem_limit_bytes=128 * 1024 * 1024,
        ),
    )(x, w_mat, scale_x, scale_w)


# device time: 74550 ns/iter; 1.0572x vs baseline; 1.0572x over previous
import os

import jax
import jax.numpy as jnp
from jax import lax
from jax.experimental import pallas as pl
from jax.experimental.pallas import tpu as pltpu

N_DEV = 8
H = 2
KVAR = os.environ.get("KVAR", "full")


def kernel(x, w_mat, scale_x, scale_w):
    m_per, k = x.shape
    _, n_total = w_mat.shape
    n_per = n_total // N_DEV
    wc = n_per // H

    def body(x_ref, w_ref, sx_ref, sw_ref, out_ref,
             wbuf, send_q, recv_q, send_s, recv_s, stage,
             wsems, osems, qsend_sems, qrecv_sems, ssend_sems, srecv_sems):
        my = lax.axis_index("i")

        scale = sx_ref[0] * sw_ref[0]
        x_q = x_ref[...].astype(jnp.float8_e4m3fn)

        order = list(range(1, N_DEV)) + [0]
        steps = [(d, h) for d in order for h in range(H)]

        out_dmas = {}

        def flush_block(slot, row_dev):
            dma = pltpu.make_async_copy(
                stage.at[slot],
                out_ref.at[pl.ds(row_dev * m_per, m_per)],
                osems.at[slot],
            )
            dma.start()
            out_dmas[slot] = dma

        def reuse_slot(slot):
            if slot in out_dmas:
                out_dmas.pop(slot).wait()

        def w_copy(d, h, slot):
            tgt = lax.rem(my + d, N_DEV)
            return pltpu.make_async_copy(
                w_ref.at[:, pl.ds(tgt * n_per + h * wc, wc)],
                wbuf.at[slot],
                wsems.at[slot],
            )

        def make_rdmas(d):
            tgt = lax.rem(my + d, N_DEV)
            data = pltpu.make_async_remote_copy(
                src_ref=send_q.at[d - 1],
                dst_ref=recv_q.at[d - 1],
                send_sem=qsend_sems.at[d - 1],
                recv_sem=qrecv_sems.at[d - 1],
                device_id=(tgt,),
                device_id_type=pl.DeviceIdType.MESH,
            )
            scl = pltpu.make_async_remote_copy(
                src_ref=send_s.at[d - 1],
                dst_ref=recv_s.at[d - 1],
                send_sem=ssend_sems.at[d - 1],
                recv_sem=srecv_sems.at[d - 1],
                device_id=(tgt,),
                device_id_type=pl.DeviceIdType.MESH,
            )
            return data, scl

        def process_recv(d):
            src_dev = lax.rem(my - d + N_DEV, N_DEV)
            data, scl = make_rdmas(d)
            data.wait_recv()
            scl.wait_recv()
            slot = d % 2
            reuse_slot(slot)
            stage[slot] = recv_q[d - 1].astype(jnp.float32) * recv_s[d - 1]
            flush_block(slot, src_dev)

        rdmas = {}
        if KVAR != "nocompute":
            w_copy(*steps[0], 0).start()
            w_copy(*steps[1], 1).start()
            for si, (d, h) in enumerate(steps):
                slot = si % 3
                if si + 2 < len(steps):
                    w_copy(*steps[si + 2], (si + 2) % 3).start()
                w_copy(d, h, slot).wait()
                wj = wbuf[slot].astype(jnp.float8_e4m3fn)
                acc = lax.dot_general(
                    x_q, wj, (((1,), (0,)), ((), ())),
                    preferred_element_type=jnp.float32)
                y = acc * scale
                z = y * (1.0 / (1.0 + jnp.exp(-jnp.clip(y, -60.0, 60.0))))
                if d == 0:
                    if h == 0:
                        reuse_slot(0)
                    stage[0, :, pl.ds(h * wc, wc)] = z
                    if h == H - 1:
                        flush_block(0, my)
                else:
                    s_h = jnp.maximum(
                        jnp.max(jnp.abs(z), axis=0, keepdims=True),
                        1e-30) * (1.0 / 127.0)
                    q = jnp.clip(jnp.round(z * (1.0 / s_h)), -127.0, 127.0)
                    send_q[d - 1, :, pl.ds(h * wc, wc)] = q.astype(jnp.int8)
                    send_s[d - 1, :, pl.ds(h * wc, wc)] = s_h
                    if h == H - 1 and KVAR == "full":
                        data, scl = make_rdmas(d)
                        data.start()
                        scl.start()
                        rdmas[d] = (data, scl)
        else:
            for d in range(1, N_DEV):
                data, scl = make_rdmas(d)
                data.start()
                scl.start()
                rdmas[d] = (data, scl)

        if KVAR != "nocomm":
            for d in range(1, N_DEV):
                process_recv(d)
            for d in rdmas:
                rdmas[d][0].wait_send()
                rdmas[d][1].wait_send()
        for slot in list(out_dmas):
            out_dmas.pop(slot).wait()

    return pl.pallas_call(
        body,
        out_shape=jax.ShapeDtypeStruct((N_DEV * m_per, n_per), jnp.float32),
        in_specs=[
            pl.BlockSpec(memory_space=pltpu.VMEM),
            pl.BlockSpec(memory_space=pl.ANY),
            pl.BlockSpec(memory_space=pltpu.SMEM),
            pl.BlockSpec(memory_space=pltpu.SMEM),
        ],
        out_specs=pl.BlockSpec(memory_space=pl.ANY),
        scratch_shapes=[
            pltpu.VMEM((3, k, wc), jnp.float32),
            pltpu.VMEM((N_DEV - 1, m_per, n_per), jnp.int8),
            pltpu.VMEM((N_DEV - 1, m_per, n_per), jnp.int8),
            pltpu.VMEM((N_DEV - 1, 1, n_per), jnp.float32),
            pltpu.VMEM((N_DEV - 1, 1, n_per), jnp.float32),
            pltpu.VMEM((2, m_per, n_per), jnp.float32),
            pltpu.SemaphoreType.DMA((3,)),
            pltpu.SemaphoreType.DMA((2,)),
            pltpu.SemaphoreType.DMA((N_DEV - 1,)),
            pltpu.SemaphoreType.DMA((N_DEV - 1,)),
            pltpu.SemaphoreType.DMA((N_DEV - 1,)),
            pltpu.SemaphoreType.DMA((N_DEV - 1,)),
        ],
        compiler_params=pltpu.CompilerParams(
            vmem_limit_bytes=128 * 1024 * 1024,
        ),
    )(x, w_mat, scale_x, scale_w)
